# Initial kernel scaffold; baseline (speedup 1.0000x reference)
import jax
import jax.numpy as jnp
from jax.experimental import pallas as pl


def kernel(primal, dual, obj_indices, obj_values, cons_indices, cons_values, obj_vector, W1, b1, W2, b2, W3, b3, delta):
    raise NotImplementedError("write your pallas kernel here")


# speedup vs baseline: 4.4479x; 4.4479x over previous
import functools

import jax
import jax.numpy as jnp
from jax import lax
from jax.experimental import pallas as pl
from jax.experimental.pallas import tpu as pltpu
from jax.experimental.pallas import tpu_sc as plsc

_N = 16384
_H = 64
_ITERS = 4
_NCORES = 2
_NSUB = 16
_NW = _NCORES * _NSUB
_LANES = 16
_KB = 128
_ROWS_PER_TILE = _N // _NSUB


def _splat_lane(v16, lane):
    idx = jnp.full((_LANES, 1), lane, dtype=jnp.int32)
    dnums = lax.GatherDimensionNumbers(
        offset_dims=(), collapsed_slice_dims=(0,), start_index_map=(0,))
    return lax.gather(v16, idx, dnums, (1,),
                      mode=lax.GatherScatterMode.PROMISE_IN_BOUNDS)


def _sc_spmm(x, cols, rows, vals):
    nnz = cols.shape[0]
    per_tile = -(-nnz // _NW)
    per_tile = -(-per_tile // _KB) * _KB
    total = per_tile * _NW
    pad = total - nnz
    cols_p = jnp.pad(cols, (0, pad))
    rows_p = jnp.pad(rows, (0, pad))
    vals_p = jnp.pad(vals, (0, pad))
    n_batches = per_tile // _KB

    mesh = plsc.VectorSubcoreMesh(core_axis_name="c", subcore_axis_name="s")

    @functools.partial(
        pl.kernel,
        out_type=jax.ShapeDtypeStruct((_NCORES, _N, _H), jnp.float32),
        mesh=mesh,
        scratch_types=[
            pltpu.VMEM((_KB,), jnp.int32),
            pltpu.VMEM((_KB,), jnp.int32),
            pltpu.VMEM((_KB,), jnp.float32),
            pltpu.VMEM((_KB, _H), jnp.float32),
            pltpu.VMEM((_KB, _H), jnp.float32),
            pltpu.VMEM_SHARED((_N, _H), jnp.float32),
            pltpu.SemaphoreType.DMA,
        ],
        compiler_params=pltpu.CompilerParams(use_tc_tiling_on_sc=False),
    )
    def spmm(x_hbm, cols_hbm, rows_hbm, vals_hbm, out_hbm,
             cbuf, rbuf, vbuf, gbuf, zbuf, acc, sem):
        cid = lax.axis_index("c")
        sid = lax.axis_index("s")
        wid = cid * _NSUB + sid

        def zrow(r, carry):
            for c in range(_H // _LANES):
                zbuf[r, pl.ds(c * _LANES, _LANES)] = jnp.zeros(
                    (_LANES,), jnp.float32)
            return carry
        lax.fori_loop(0, _KB, zrow, 0)

        def zchunk(s, carry):
            pltpu.sync_copy(
                zbuf, acc.at[pl.ds(sid * _ROWS_PER_TILE + s * _KB, _KB)])
            return carry
        lax.fori_loop(0, _ROWS_PER_TILE // _KB, zchunk, 0)
        plsc.subcore_barrier()

        base0 = wid * per_tile

        def batch(b, carry):
            off = base0 + b * _KB
            pltpu.sync_copy(cols_hbm.at[pl.ds(off, _KB)], cbuf)
            pltpu.sync_copy(rows_hbm.at[pl.ds(off, _KB)], rbuf)
            pltpu.sync_copy(vals_hbm.at[pl.ds(off, _KB)], vbuf)
            pltpu.async_copy(x_hbm.at[cbuf], gbuf, sem).wait()

            def group(g, carry2):
                v16 = vbuf[pl.ds(g * _LANES, _LANES)]
                for l in range(_LANES):
                    e = g * _LANES + l
                    s = _splat_lane(v16, l)
                    for c in range(_H // _LANES):
                        sl = pl.ds(c * _LANES, _LANES)
                        gbuf[e, sl] = gbuf[e, sl] * s
                return carry2
            lax.fori_loop(0, _KB // _LANES, group, 0)

            pltpu.sync_copy(gbuf, acc.at[rbuf], add=True)
            return carry
        lax.fori_loop(0, n_batches, batch, 0)
        plsc.subcore_barrier()

        def wout(s, carry):
            sl = pl.ds(sid * _ROWS_PER_TILE + s * _KB, _KB)
            pltpu.sync_copy(acc.at[sl], out_hbm.at[cid, sl])
            return carry
        lax.fori_loop(0, _ROWS_PER_TILE // _KB, wout, 0)

    return spmm(x, cols_p, rows_p, vals_p)


def _tc_update(primal, sp1p, sp2p, obj_vector, W1i, W2i, W3i,
               b1r, b2r, b3r, d2):
    BR = 2048
    grid = (_N // BR,)
    dn = (((1,), (1,)), ((), ()))

    def body(x_ref, s1_ref, s2_ref, obj_ref, w1_ref, w2_ref, w3_ref,
             b1_ref, b2_ref, b3_ref, d_ref, o_ref):
        x = x_ref[...]
        t1 = lax.dot_general(x, w1_ref[...], dn,
                             preferred_element_type=jnp.float32) + b1_ref[...]
        s1 = s1_ref[0] + s1_ref[1]
        t2 = lax.dot_general(s1, w2_ref[...], dn,
                             preferred_element_type=jnp.float32) + b2_ref[...]
        s2 = s2_ref[0] + s2_ref[1]
        t3 = lax.dot_general(s2, w3_ref[...], dn,
                             preferred_element_type=jnp.float32) + b3_ref[...]
        d = d_ref[0, 0]
        z = t1 - d * (t2 + obj_ref[...] - t3)
        o_ref[...] = jnp.where(z >= 0, z, 0.01 * z)

    row_spec = pl.BlockSpec((BR, _H), lambda i: (i, 0))
    part_spec = pl.BlockSpec((2, BR, _H), lambda i: (0, i, 0))
    w_spec = pl.BlockSpec((_H, _H), lambda i: (0, 0))
    b_spec = pl.BlockSpec((1, _H), lambda i: (0, 0))
    d_spec = pl.BlockSpec((1, 1), lambda i: (0, 0))
    return pl.pallas_call(
        body,
        grid=grid,
        in_specs=[row_spec, part_spec, part_spec, row_spec,
                  w_spec, w_spec, w_spec, b_spec, b_spec, b_spec, d_spec],
        out_specs=row_spec,
        out_shape=jax.ShapeDtypeStruct((_N, _H), jnp.float32),
    )(primal, sp1p, sp2p, obj_vector, W1i, W2i, W3i, b1r, b2r, b3r, d2)


def kernel(primal, dual, obj_indices, obj_values, cons_indices, cons_values,
           obj_vector, W1, b1, W2, b2, W3, b3, delta):
    obj_rows = obj_indices[0]
    obj_cols = obj_indices[1]
    sp2p = _sc_spmm(dual, cons_indices[0], cons_indices[1], cons_values)
    d2 = jnp.reshape(delta, (1, 1))
    b1r = jnp.reshape(b1, (_ITERS, 1, _H))
    b2r = jnp.reshape(b2, (_ITERS, 1, _H))
    b3r = jnp.reshape(b3, (_ITERS, 1, _H))
    for i in range(_ITERS):
        sp1p = _sc_spmm(primal, obj_cols, obj_rows, obj_values)
        primal = _tc_update(primal, sp1p, sp2p, obj_vector,
                            W1[i], W2[i], W3[i],
                            b1r[i], b2r[i], b3r[i], d2)
    return primal


# speedup vs baseline: 5.4335x; 1.2216x over previous
import functools

import jax
import jax.numpy as jnp
from jax import lax
from jax.experimental import pallas as pl
from jax.experimental.pallas import tpu as pltpu
from jax.experimental.pallas import tpu_sc as plsc

_N = 16384
_H = 64
_ITERS = 4
_NCORES = 2
_NSUB = 16
_NW = _NCORES * _NSUB
_LANES = 16
_KB = 128
_NSLOT = 4
_ROWS_PER_TILE = _N // _NSUB
_HC = _H // _LANES
_ZROWS = 64


def _splat_lane(v16, lane):
    idx = jnp.full((_LANES, 1), lane, dtype=jnp.int32)
    dnums = lax.GatherDimensionNumbers(
        offset_dims=(), collapsed_slice_dims=(0,), start_index_map=(0,))
    return lax.gather(v16, idx, dnums, (1,),
                      mode=lax.GatherScatterMode.PROMISE_IN_BOUNDS)


def _sc_spmm(x, cols, rows, vals):
    nnz = cols.shape[0]
    chunk = _KB * _NSLOT
    per_tile = -(-nnz // _NW)
    per_tile = -(-per_tile // chunk) * chunk
    total = per_tile * _NW
    pad = total - nnz
    cols_p = jnp.pad(cols, (0, pad))
    rows_p = jnp.pad(rows, (0, pad))
    vals_p = jnp.pad(vals, (0, pad))
    n_batches = per_tile // _KB
    n_rounds = n_batches // _NSLOT

    mesh = plsc.VectorSubcoreMesh(core_axis_name="c", subcore_axis_name="s")

    @functools.partial(
        pl.kernel,
        out_type=jax.ShapeDtypeStruct((_NCORES, _N, _H), jnp.float32),
        mesh=mesh,
        scratch_types=[
            pltpu.VMEM((per_tile,), jnp.int32),
            pltpu.VMEM((per_tile,), jnp.int32),
            pltpu.VMEM((per_tile,), jnp.float32),
            [pltpu.VMEM((_KB,), jnp.int32) for _ in range(_NSLOT)],
            [pltpu.VMEM((_KB,), jnp.int32) for _ in range(_NSLOT)],
            [pltpu.VMEM((_KB, _H), jnp.float32) for _ in range(_NSLOT)],
            pltpu.VMEM((_ZROWS, _H), jnp.float32),
            pltpu.VMEM_SHARED((_N, _H), jnp.float32),
            [pltpu.SemaphoreType.DMA for _ in range(_NSLOT)],
            [pltpu.SemaphoreType.DMA for _ in range(_NSLOT)],
        ],
        compiler_params=pltpu.CompilerParams(use_tc_tiling_on_sc=False),
    )
    def spmm(x_hbm, cols_hbm, rows_hbm, vals_hbm, out_hbm,
             ecols, erows, evals, cbufs, rbufs, gbufs, zbuf, acc,
             gsems, ssems):
        cid = lax.axis_index("c")
        sid = lax.axis_index("s")
        wid = cid * _NSUB + sid
        base0 = wid * per_tile

        pre_c = pltpu.async_copy(
            cols_hbm.at[pl.ds(base0, per_tile)], ecols, gsems[0])
        pre_r = pltpu.async_copy(
            rows_hbm.at[pl.ds(base0, per_tile)], erows, gsems[1])
        pre_v = pltpu.async_copy(
            vals_hbm.at[pl.ds(base0, per_tile)], evals, gsems[2])

        def zrow(r, carry):
            for c in range(_HC):
                zbuf[r, pl.ds(c * _LANES, _LANES)] = jnp.zeros(
                    (_LANES,), jnp.float32)
            return carry
        lax.fori_loop(0, _ZROWS, zrow, 0)

        def fill_cbuf(k, b):
            off = b * _KB
            for j in range(_KB // _LANES):
                cbufs[k][pl.ds(j * _LANES, _LANES)] = (
                    ecols[pl.ds(off + j * _LANES, _LANES)])

        def gather_start(k, b):
            fill_cbuf(k, b)
            pltpu.async_copy(x_hbm.at[cbufs[k]], gbufs[k], gsems[k])

        def gather_wait(k):
            pltpu.make_async_copy(x_hbm.at[cbufs[k]], gbufs[k],
                                  gsems[k]).wait()

        def scale(k, b):
            boff = b * _KB

            def group(g, carry):
                v16 = evals[pl.ds(boff + g * _LANES, _LANES)]
                for l in range(_LANES):
                    e = g * _LANES + l
                    s = _splat_lane(v16, l)
                    for c in range(_HC):
                        sl = pl.ds(c * _LANES, _LANES)
                        gbufs[k][e, sl] = gbufs[k][e, sl] * s
                return carry
            lax.fori_loop(0, _KB // _LANES, group, 0)

        def scatter_start(k, b):
            off = b * _KB
            for j in range(_KB // _LANES):
                rbufs[k][pl.ds(j * _LANES, _LANES)] = (
                    erows[pl.ds(off + j * _LANES, _LANES)])
            pltpu.sync_copy(gbufs[k], acc.at[rbufs[k]], add=True)

        def scatter_wait(k):
            pass

        pre_c.wait()
        for k in range(_NSLOT):
            gather_start(k, k)

        def zchunk(s, carry):
            pltpu.sync_copy(
                zbuf,
                acc.at[pl.ds(sid * _ROWS_PER_TILE + s * _ZROWS, _ZROWS)])
            return carry
        lax.fori_loop(0, _ROWS_PER_TILE // _ZROWS, zchunk, 0)
        pre_r.wait()
        pre_v.wait()
        plsc.subcore_barrier()

        def round_body(i, carry):
            b0 = i * _NSLOT
            for k in range(_NSLOT):
                b = b0 + k
                gather_wait(k)
                scale(k, b)
                scatter_start(k, b)
                if k > 0:
                    scatter_wait(k - 1)
                    nxt = b0 + _NSLOT + k - 1

                    @pl.when(nxt < n_batches)
                    def _():
                        gather_start(k - 1, nxt)
            scatter_wait(_NSLOT - 1)
            nxt = b0 + 2 * _NSLOT - 1

            @pl.when(nxt < n_batches)
            def _():
                gather_start(_NSLOT - 1, nxt)
            return carry
        lax.fori_loop(0, n_rounds, round_body, 0)
        plsc.subcore_barrier()

        def wout(s, carry):
            sl = pl.ds(sid * _ROWS_PER_TILE + s * _KB, _KB)
            pltpu.sync_copy(acc.at[sl], out_hbm.at[cid, sl])
            return carry
        lax.fori_loop(0, _ROWS_PER_TILE // _KB, wout, 0)

    return spmm(x, cols_p, rows_p, vals_p)


def _tc_update(primal, sp1p, sp2p, obj_vector, W1i, W2i, W3i,
               b1r, b2r, b3r, d2):
    BR = 2048
    grid = (_N // BR,)
    dn = (((1,), (1,)), ((), ()))

    def body(x_ref, s1_ref, s2_ref, obj_ref, w1_ref, w2_ref, w3_ref,
             b1_ref, b2_ref, b3_ref, d_ref, o_ref):
        x = x_ref[...]
        t1 = lax.dot_general(x, w1_ref[...], dn,
                             preferred_element_type=jnp.float32) + b1_ref[...]
        s1 = s1_ref[0] + s1_ref[1]
        t2 = lax.dot_general(s1, w2_ref[...], dn,
                             preferred_element_type=jnp.float32) + b2_ref[...]
        s2 = s2_ref[0] + s2_ref[1]
        t3 = lax.dot_general(s2, w3_ref[...], dn,
                             preferred_element_type=jnp.float32) + b3_ref[...]
        d = d_ref[0, 0]
        z = t1 - d * (t2 + obj_ref[...] - t3)
        o_ref[...] = jnp.where(z >= 0, z, 0.01 * z)

    row_spec = pl.BlockSpec((BR, _H), lambda i: (i, 0))
    part_spec = pl.BlockSpec((2, BR, _H), lambda i: (0, i, 0))
    w_spec = pl.BlockSpec((_H, _H), lambda i: (0, 0))
    b_spec = pl.BlockSpec((1, _H), lambda i: (0, 0))
    d_spec = pl.BlockSpec((1, 1), lambda i: (0, 0))
    return pl.pallas_call(
        body,
        grid=grid,
        in_specs=[row_spec, part_spec, part_spec, row_spec,
                  w_spec, w_spec, w_spec, b_spec, b_spec, b_spec, d_spec],
        out_specs=row_spec,
        out_shape=jax.ShapeDtypeStruct((_N, _H), jnp.float32),
    )(primal, sp1p, sp2p, obj_vector, W1i, W2i, W3i, b1r, b2r, b3r, d2)


def kernel(primal, dual, obj_indices, obj_values, cons_indices, cons_values,
           obj_vector, W1, b1, W2, b2, W3, b3, delta):
    obj_rows = obj_indices[0]
    obj_cols = obj_indices[1]
    sp2p = _sc_spmm(dual, cons_indices[0], cons_indices[1], cons_values)
    d2 = jnp.reshape(delta, (1, 1))
    b1r = jnp.reshape(b1, (_ITERS, 1, _H))
    b2r = jnp.reshape(b2, (_ITERS, 1, _H))
    b3r = jnp.reshape(b3, (_ITERS, 1, _H))
    for i in range(_ITERS):
        sp1p = _sc_spmm(primal, obj_cols, obj_rows, obj_values)
        primal = _tc_update(primal, sp1p, sp2p, obj_vector,
                            W1[i], W2[i], W3[i],
                            b1r[i], b2r[i], b3r[i], d2)
    return primal


# speedup vs baseline: 5.5591x; 1.0231x over previous
import functools

import jax
import jax.numpy as jnp
from jax import lax
from jax.experimental import pallas as pl
from jax.experimental.pallas import tpu as pltpu
from jax.experimental.pallas import tpu_sc as plsc

_N = 16384
_H = 64
_ITERS = 4
_NCORES = 2
_NSUB = 16
_NW = _NCORES * _NSUB
_LANES = 16
_KB = 128
_NSLOT = 4
_ROWS_PER_TILE = _N // _NSUB
_HC = _H // _LANES
_ZROWS = 64


def _splat_lane(v16, lane):
    idx = jnp.full((_LANES, 1), lane, dtype=jnp.int32)
    dnums = lax.GatherDimensionNumbers(
        offset_dims=(), collapsed_slice_dims=(0,), start_index_map=(0,))
    return lax.gather(v16, idx, dnums, (1,),
                      mode=lax.GatherScatterMode.PROMISE_IN_BOUNDS)


def _sc_spmm(x, cols, rows, vals):
    nnz = cols.shape[0]
    chunk = _KB * _NSLOT
    per_tile = -(-nnz // _NW)
    per_tile = -(-per_tile // chunk) * chunk
    total = per_tile * _NW
    pad = total - nnz
    cols_p = jnp.pad(cols, (0, pad))
    rows_p = jnp.pad(rows, (0, pad))
    vals_p = jnp.pad(vals, (0, pad))
    n_batches = per_tile // _KB
    n_rounds = n_batches // _NSLOT

    mesh = plsc.VectorSubcoreMesh(core_axis_name="c", subcore_axis_name="s")

    @functools.partial(
        pl.kernel,
        out_type=jax.ShapeDtypeStruct((_NCORES, _N, _H), jnp.float32),
        mesh=mesh,
        scratch_types=[
            pltpu.VMEM((per_tile,), jnp.int32),
            pltpu.VMEM((per_tile,), jnp.int32),
            pltpu.VMEM((per_tile,), jnp.float32),
            [pltpu.VMEM((_KB,), jnp.int32) for _ in range(_NSLOT)],
            [pltpu.VMEM((_KB,), jnp.int32) for _ in range(_NSLOT)],
            [pltpu.VMEM((_KB, _H), jnp.float32) for _ in range(_NSLOT)],
            pltpu.VMEM((_ZROWS, _H), jnp.float32),
            pltpu.VMEM_SHARED((_N, _H), jnp.float32),
            [pltpu.SemaphoreType.DMA for _ in range(_NSLOT)],
            [pltpu.SemaphoreType.DMA for _ in range(_NSLOT)],
        ],
        compiler_params=pltpu.CompilerParams(use_tc_tiling_on_sc=False),
    )
    def spmm(x_hbm, cols_hbm, rows_hbm, vals_hbm, out_hbm,
             ecols, erows, evals, cbufs, rbufs, gbufs, zbuf, acc,
             gsems, ssems):
        cid = lax.axis_index("c")
        sid = lax.axis_index("s")
        wid = cid * _NSUB + sid
        base0 = wid * per_tile

        pre_c = pltpu.async_copy(
            cols_hbm.at[pl.ds(base0, per_tile)], ecols, gsems[0])
        pre_r = pltpu.async_copy(
            rows_hbm.at[pl.ds(base0, per_tile)], erows, gsems[1])
        pre_v = pltpu.async_copy(
            vals_hbm.at[pl.ds(base0, per_tile)], evals, gsems[2])

        def zrow(r, carry):
            for c in range(_HC):
                zbuf[r, pl.ds(c * _LANES, _LANES)] = jnp.zeros(
                    (_LANES,), jnp.float32)
            return carry
        lax.fori_loop(0, _ZROWS, zrow, 0)

        def fill_cbuf(k, b):
            off = b * _KB
            for j in range(_KB // _LANES):
                cbufs[k][pl.ds(j * _LANES, _LANES)] = (
                    ecols[pl.ds(off + j * _LANES, _LANES)])

        def gather_start(k, b):
            fill_cbuf(k, b)
            pltpu.async_copy(x_hbm.at[cbufs[k]], gbufs[k], gsems[k])

        def gather_wait(k):
            pltpu.make_async_copy(x_hbm.at[cbufs[k]], gbufs[k],
                                  gsems[k]).wait()

        def scale(k, b):
            boff = b * _KB

            def group(g, carry):
                v16 = evals[pl.ds(boff + g * _LANES, _LANES)]
                for l in range(_LANES):
                    e = g * _LANES + l
                    s = _splat_lane(v16, l)
                    for c in range(_HC):
                        sl = pl.ds(c * _LANES, _LANES)
                        gbufs[k][e, sl] = gbufs[k][e, sl] * s
                return carry
            lax.fori_loop(0, _KB // _LANES, group, 0)

        def scatter_start(k, b):
            off = b * _KB
            for j in range(_KB // _LANES):
                rbufs[k][pl.ds(j * _LANES, _LANES)] = (
                    erows[pl.ds(off + j * _LANES, _LANES)])
            pltpu.async_copy(gbufs[k], acc.at[rbufs[k]], ssems[k], add=True)

        def scatter_wait(k):
            pltpu.make_async_copy(gbufs[k], acc.at[rbufs[k]],
                                  ssems[k]).wait()

        pre_c.wait()
        for k in range(_NSLOT):
            gather_start(k, k)

        def zchunk(s, carry):
            pltpu.sync_copy(
                zbuf,
                acc.at[pl.ds(sid * _ROWS_PER_TILE + s * _ZROWS, _ZROWS)])
            return carry
        lax.fori_loop(0, _ROWS_PER_TILE // _ZROWS, zchunk, 0)
        pre_r.wait()
        pre_v.wait()
        plsc.subcore_barrier()

        def round_body(i, carry):
            b0 = i * _NSLOT
            for k in range(_NSLOT):
                b = b0 + k
                gather_wait(k)
                scale(k, b)
                scatter_start(k, b)
                if k > 0:
                    scatter_wait(k - 1)
                    nxt = b0 + _NSLOT + k - 1

                    @pl.when(nxt < n_batches)
                    def _():
                        gather_start(k - 1, nxt)
            scatter_wait(_NSLOT - 1)
            nxt = b0 + 2 * _NSLOT - 1

            @pl.when(nxt < n_batches)
            def _():
                gather_start(_NSLOT - 1, nxt)
            return carry
        lax.fori_loop(0, n_rounds, round_body, 0)
        plsc.subcore_barrier()

        def wout(s, carry):
            sl = pl.ds(sid * _ROWS_PER_TILE + s * _KB, _KB)
            pltpu.sync_copy(acc.at[sl], out_hbm.at[cid, sl])
            return carry
        lax.fori_loop(0, _ROWS_PER_TILE // _KB, wout, 0)

    return spmm(x, cols_p, rows_p, vals_p)


def _tc_update(primal, sp1p, sp2p, obj_vector, W1i, W2i, W3i,
               b1r, b2r, b3r, d2):
    BR = 2048
    grid = (_N // BR,)
    dn = (((1,), (1,)), ((), ()))

    def body(x_ref, s1_ref, s2_ref, obj_ref, w1_ref, w2_ref, w3_ref,
             b1_ref, b2_ref, b3_ref, d_ref, o_ref):
        x = x_ref[...]
        t1 = lax.dot_general(x, w1_ref[...], dn,
                             preferred_element_type=jnp.float32) + b1_ref[...]
        s1 = s1_ref[0] + s1_ref[1]
        t2 = lax.dot_general(s1, w2_ref[...], dn,
                             preferred_element_type=jnp.float32) + b2_ref[...]
        s2 = s2_ref[0] + s2_ref[1]
        t3 = lax.dot_general(s2, w3_ref[...], dn,
                             preferred_element_type=jnp.float32) + b3_ref[...]
        d = d_ref[0, 0]
        z = t1 - d * (t2 + obj_ref[...] - t3)
        o_ref[...] = jnp.where(z >= 0, z, 0.01 * z)

    row_spec = pl.BlockSpec((BR, _H), lambda i: (i, 0))
    part_spec = pl.BlockSpec((2, BR, _H), lambda i: (0, i, 0))
    w_spec = pl.BlockSpec((_H, _H), lambda i: (0, 0))
    b_spec = pl.BlockSpec((1, _H), lambda i: (0, 0))
    d_spec = pl.BlockSpec((1, 1), lambda i: (0, 0))
    return pl.pallas_call(
        body,
        grid=grid,
        in_specs=[row_spec, part_spec, part_spec, row_spec,
                  w_spec, w_spec, w_spec, b_spec, b_spec, b_spec, d_spec],
        out_specs=row_spec,
        out_shape=jax.ShapeDtypeStruct((_N, _H), jnp.float32),
    )(primal, sp1p, sp2p, obj_vector, W1i, W2i, W3i, b1r, b2r, b3r, d2)


def kernel(primal, dual, obj_indices, obj_values, cons_indices, cons_values,
           obj_vector, W1, b1, W2, b2, W3, b3, delta):
    obj_rows = obj_indices[0]
    obj_cols = obj_indices[1]
    sp2p = _sc_spmm(dual, cons_indices[0], cons_indices[1], cons_values)
    d2 = jnp.reshape(delta, (1, 1))
    b1r = jnp.reshape(b1, (_ITERS, 1, _H))
    b2r = jnp.reshape(b2, (_ITERS, 1, _H))
    b3r = jnp.reshape(b3, (_ITERS, 1, _H))
    for i in range(_ITERS):
        sp1p = _sc_spmm(primal, obj_cols, obj_rows, obj_values)
        primal = _tc_update(primal, sp1p, sp2p, obj_vector,
                            W1[i], W2[i], W3[i],
                            b1r[i], b2r[i], b3r[i], d2)
    return primal


# speedup vs baseline: 6.5173x; 1.1724x over previous
import functools

import jax
import jax.numpy as jnp
from jax import lax
from jax.experimental import pallas as pl
from jax.experimental.pallas import tpu as pltpu
from jax.experimental.pallas import tpu_sc as plsc

_N = 16384
_H = 64
_ITERS = 4
_NCORES = 2
_NSUB = 16
_NW = _NCORES * _NSUB
_LANES = 16
_KB = 128
_NGS = 4
_NSS = 2
_SB = _KB // _NSS
_ROWS_PER_TILE = _N // _NSUB
_HC = _H // _LANES
_ZROWS = 64

_GATHER_DNUMS = lax.GatherDimensionNumbers(
    offset_dims=(), collapsed_slice_dims=(0,), start_index_map=(0,))


def _splat_lane(v16, lane):
    idx = jnp.full((_LANES, 1), lane, dtype=jnp.int32)
    return lax.gather(v16, idx, _GATHER_DNUMS, (1,),
                      mode=lax.GatherScatterMode.PROMISE_IN_BOUNDS)


def _sc_spmm(xbf, cols, rows, vals):
    nnz = cols.shape[0]
    chunk = _KB * _NGS
    per_tile = -(-nnz // _NW)
    per_tile = -(-per_tile // chunk) * chunk
    total = per_tile * _NW
    pad = total - nnz
    cols_p = jnp.pad(cols, (0, pad))
    rows_p = jnp.pad(rows, (0, pad))
    vals_p = jnp.pad(vals, (0, pad))
    n_batches = per_tile // _KB
    n_rounds = n_batches // _NGS

    mesh = plsc.VectorSubcoreMesh(core_axis_name="c", subcore_axis_name="s")

    @functools.partial(
        pl.kernel,
        out_type=jax.ShapeDtypeStruct((_NCORES, _N, _H), jnp.float32),
        mesh=mesh,
        scratch_types=[
            [pltpu.VMEM((_KB,), jnp.int32) for _ in range(_NGS)],
            [pltpu.VMEM((_KB,), jnp.int32) for _ in range(_NGS)],
            [pltpu.VMEM((_KB,), jnp.float32) for _ in range(_NGS)],
            [pltpu.VMEM((_SB,), jnp.int32) for _ in range(_NSS)],
            [pltpu.VMEM((_KB, _H), jnp.bfloat16) for _ in range(_NGS)],
            [pltpu.VMEM((_SB, _H), jnp.float32) for _ in range(_NSS)],
            pltpu.VMEM((_ZROWS, _H), jnp.float32),
            pltpu.VMEM_SHARED((_N, _H), jnp.float32),
            pltpu.VMEM_SHARED((_N, _H), jnp.bfloat16),
            [pltpu.SemaphoreType.DMA for _ in range(_NGS)],
            [pltpu.SemaphoreType.DMA for _ in range(_NGS)],
            [pltpu.SemaphoreType.DMA for _ in range(_NSS)],
        ],
        compiler_params=pltpu.CompilerParams(use_tc_tiling_on_sc=False),
    )
    def spmm(xbf_hbm, cols_hbm, rows_hbm, vals_hbm, out_hbm,
             cbufs, rbufs, vbufs, sbufs, gbufs, fbufs, zbuf, acc, xs,
             esems, gsems, ssems):
        cid = lax.axis_index("c")
        sid = lax.axis_index("s")
        wid = cid * _NSUB + sid
        base0 = wid * per_tile

        def estart(k, b):
            off = base0 + b * _KB
            pltpu.async_copy(cols_hbm.at[pl.ds(off, _KB)], cbufs[k], esems[k])
            pltpu.async_copy(rows_hbm.at[pl.ds(off, _KB)], rbufs[k], esems[k])
            pltpu.async_copy(vals_hbm.at[pl.ds(off, _KB)], vbufs[k], esems[k])

        def ewait(k, b):
            off = base0 + b * _KB
            pltpu.make_async_copy(
                cols_hbm.at[pl.ds(off, _KB)], cbufs[k], esems[k]).wait()
            pltpu.make_async_copy(
                rows_hbm.at[pl.ds(off, _KB)], rbufs[k], esems[k]).wait()
            pltpu.make_async_copy(
                vals_hbm.at[pl.ds(off, _KB)], vbufs[k], esems[k]).wait()

        def gather_start(k):
            pltpu.async_copy(xs.at[cbufs[k]], gbufs[k], gsems[k])

        def gather_wait(k):
            pltpu.make_async_copy(xs.at[cbufs[k]], gbufs[k],
                                  gsems[k]).wait()

        def scale_half(k, h):
            def group(q, carry):
                goff = h * _SB + q * _LANES
                v16 = vbufs[k][pl.ds(goff, _LANES)]
                for l in range(_LANES):
                    e = goff + l
                    s = _splat_lane(v16, l)
                    for c in range(_HC):
                        sl = pl.ds(c * _LANES, _LANES)
                        fbufs[h][q * _LANES + l, sl] = (
                            gbufs[k][e, sl].astype(jnp.float32) * s)
                return carry
            lax.fori_loop(0, _SB // _LANES, group, 0)

        def scatter_start(k, h):
            for j in range(_SB // _LANES):
                sbufs[h][pl.ds(j * _LANES, _LANES)] = (
                    rbufs[k][pl.ds(h * _SB + j * _LANES, _LANES)])
            pltpu.async_copy(fbufs[h], acc.at[sbufs[h]], ssems[h], add=True)

        def scatter_wait(h):
            pltpu.make_async_copy(fbufs[h], acc.at[sbufs[h]],
                                  ssems[h]).wait()

        for k in range(_NGS):
            estart(k, k)

        xb = sid * _ROWS_PER_TILE

        def xchunk(t, carry):
            sl = pl.ds(xb + t * _KB, _KB)
            pltpu.sync_copy(xbf_hbm.at[sl], xs.at[sl])
            return carry
        lax.fori_loop(0, _ROWS_PER_TILE // _KB, xchunk, 0)

        def zrow(r, carry):
            for c in range(_HC):
                zbuf[r, pl.ds(c * _LANES, _LANES)] = jnp.zeros(
                    (_LANES,), jnp.float32)
            return carry
        lax.fori_loop(0, _ZROWS, zrow, 0)

        def zchunk(s, carry):
            pltpu.sync_copy(
                zbuf,
                acc.at[pl.ds(sid * _ROWS_PER_TILE + s * _ZROWS, _ZROWS)])
            return carry
        lax.fori_loop(0, _ROWS_PER_TILE // _ZROWS, zchunk, 0)

        plsc.subcore_barrier()

        ewait(0, 0)
        gather_start(0)

        def body(b, k, first):
            k1 = (k + 1) % _NGS

            @pl.when(b + 1 < n_batches)
            def _():
                ewait(k1, b + 1)
                gather_start(k1)
            gather_wait(k)
            for h in range(_NSS):
                if not first:
                    scatter_wait(h)
                scale_half(k, h)
                scatter_start(k, h)

            @pl.when(b + _NGS < n_batches)
            def _():
                estart(k, b + _NGS)

        body(0, 0, True)

        def round_body(i, carry):
            b0 = i * _NGS
            for k in range(_NGS):
                b = b0 + k

                @pl.when(b > 0)
                def _():
                    body(b, k, False)
            return carry
        lax.fori_loop(0, n_rounds, round_body, 0)
        for h in range(_NSS):
            scatter_wait(h)
        plsc.subcore_barrier()

        def wout(s, carry):
            sl = pl.ds(sid * _ROWS_PER_TILE + s * _KB, _KB)
            pltpu.sync_copy(acc.at[sl], out_hbm.at[cid, sl])
            return carry
        lax.fori_loop(0, _ROWS_PER_TILE // _KB, wout, 0)

    return spmm(xbf, cols_p, rows_p, vals_p)


def _tc_update(primal, sp1p, sp2p, obj_vector, W1i, W2i, W3i,
               b1r, b2r, b3r, d2):
    BR = 2048
    grid = (_N // BR,)
    dn = (((1,), (1,)), ((), ()))

    def body(x_ref, s1_ref, s2_ref, obj_ref, w1_ref, w2_ref, w3_ref,
             b1_ref, b2_ref, b3_ref, d_ref, o_ref):
        x = x_ref[...]
        t1 = lax.dot_general(x, w1_ref[...], dn,
                             preferred_element_type=jnp.float32) + b1_ref[...]
        s1 = s1_ref[0] + s1_ref[1]
        t2 = lax.dot_general(s1, w2_ref[...], dn,
                             preferred_element_type=jnp.float32) + b2_ref[...]
        s2 = s2_ref[0] + s2_ref[1]
        t3 = lax.dot_general(s2, w3_ref[...], dn,
                             preferred_element_type=jnp.float32) + b3_ref[...]
        d = d_ref[0, 0]
        z = t1 - d * (t2 + obj_ref[...] - t3)
        o_ref[...] = jnp.where(z >= 0, z, 0.01 * z)

    row_spec = pl.BlockSpec((BR, _H), lambda i: (i, 0))
    part_spec = pl.BlockSpec((2, BR, _H), lambda i: (0, i, 0))
    w_spec = pl.BlockSpec((_H, _H), lambda i: (0, 0))
    b_spec = pl.BlockSpec((1, _H), lambda i: (0, 0))
    d_spec = pl.BlockSpec((1, 1), lambda i: (0, 0))
    return pl.pallas_call(
        body,
        grid=grid,
        in_specs=[row_spec, part_spec, part_spec, row_spec,
                  w_spec, w_spec, w_spec, b_spec, b_spec, b_spec, d_spec],
        out_specs=row_spec,
        out_shape=jax.ShapeDtypeStruct((_N, _H), jnp.float32),
    )(primal, sp1p, sp2p, obj_vector, W1i, W2i, W3i, b1r, b2r, b3r, d2)


def kernel(primal, dual, obj_indices, obj_values, cons_indices, cons_values,
           obj_vector, W1, b1, W2, b2, W3, b3, delta):
    obj_rows = obj_indices[0]
    obj_cols = obj_indices[1]
    sp2p = _sc_spmm(dual.astype(jnp.bfloat16),
                    cons_indices[0], cons_indices[1], cons_values)
    d2 = jnp.reshape(delta, (1, 1))
    b1r = jnp.reshape(b1, (_ITERS, 1, _H))
    b2r = jnp.reshape(b2, (_ITERS, 1, _H))
    b3r = jnp.reshape(b3, (_ITERS, 1, _H))
    for i in range(_ITERS):
        sp1p = _sc_spmm(primal.astype(jnp.bfloat16),
                        obj_cols, obj_rows, obj_values)
        primal = _tc_update(primal, sp1p, sp2p, obj_vector,
                            W1[i], W2[i], W3[i],
                            b1r[i], b2r[i], b3r[i], d2)
    return primal
